# baseline (speedup 1.0000x reference)
import functools

import jax
import jax.numpy as jnp
from jax import lax
from jax.experimental import pallas as pl
from jax.experimental.pallas import tpu as pltpu
from jax.experimental.pallas import tpu_sc as plsc

_NC = 2
_NS = 16
_CH = 32
_NBUF = 3
_TC_CHUNK = 512


def _tc_body(t_ref, o_ref):
    o_ref[...] = jnp.broadcast_to(t_ref[...][None], o_ref.shape)


def _tc_copy(table, B, S, D):
    return pl.pallas_call(
        _tc_body,
        grid=(S // _TC_CHUNK,),
        in_specs=[pl.BlockSpec((_TC_CHUNK, D), lambda i: (i, 0))],
        out_specs=pl.BlockSpec((B, _TC_CHUNK, D), lambda i: (0, i, 0)),
        out_shape=jax.ShapeDtypeStruct((B, S, D), table.dtype),
    )(table)


def _sc_copy(table, B, S, D):
    NW = _NC * _NS
    rows = S // NW
    nchunk = rows // _CH
    mesh = plsc.VectorSubcoreMesh(core_axis_name="c", subcore_axis_name="s")

    @functools.partial(
        pl.kernel,
        out_type=jax.ShapeDtypeStruct((B, S, D), table.dtype),
        mesh=mesh,
        scratch_types=(
            [pltpu.VMEM((_CH, D), table.dtype) for _ in range(_NBUF)]
            + [pltpu.SemaphoreType.DMA, pltpu.SemaphoreType.DMA]
        ),
    )
    def body(table_hbm, out_hbm, *rest):
        bufs, (in_sem, out_sem) = list(rest[:_NBUF]), rest[_NBUF:]
        wid = lax.axis_index("s") * _NC + lax.axis_index("c")
        base = wid * rows

        def start_in(i):
            return pltpu.async_copy(
                table_hbm.at[pl.ds(base + i * _CH, _CH)],
                bufs[i % _NBUF], in_sem)

        in_copies = [None] * nchunk
        out_copies = [None] * nchunk
        drained = [False] * nchunk
        in_copies[0] = start_in(0)
        for i in range(nchunk):
            in_copies[i].wait()
            out_copies[i] = [
                pltpu.async_copy(
                    bufs[i % _NBUF],
                    out_hbm.at[b].at[pl.ds(base + i * _CH, _CH)],
                    out_sem)
                for b in range(B)
            ]
            if i + 1 < nchunk:
                prev_user = i + 1 - _NBUF
                if prev_user >= 0:
                    for c in out_copies[prev_user]:
                        c.wait()
                    drained[prev_user] = True
                in_copies[i + 1] = start_in(i + 1)
        for i in range(nchunk):
            if not drained[i]:
                for c in out_copies[i]:
                    c.wait()

    return body(table)


def kernel(x, table):
    B, S, D = x.shape
    b_tc = B // 2
    out_tc = _tc_copy(table, b_tc, S, D)
    out_sc = _sc_copy(table, B - b_tc, S, D)
    return jnp.concatenate([out_tc, out_sc], axis=0)


# speedup vs baseline: 2.2096x; 2.2096x over previous
import functools

import jax
import jax.numpy as jnp
from jax import lax
from jax.experimental import pallas as pl
from jax.experimental.pallas import tpu as pltpu
from jax.experimental.pallas import tpu_sc as plsc

_NC = 2
_NS = 16
_CH = 32
_NBUF = 3
_TC_CHUNK = 512


def _tc_body(t_ref, o_ref):
    o_ref[...] = jnp.broadcast_to(t_ref[...][None], o_ref.shape)


def _tc_copy(table, B, S, D):
    return pl.pallas_call(
        _tc_body,
        grid=(S // _TC_CHUNK,),
        in_specs=[pl.BlockSpec((_TC_CHUNK, D), lambda i: (i, 0))],
        out_specs=pl.BlockSpec((B, _TC_CHUNK, D), lambda i: (0, i, 0)),
        out_shape=jax.ShapeDtypeStruct((B, S, D), table.dtype),
    )(table)


def _sc_copy(table, B, S, D):
    NW = _NC * _NS
    rows = S // NW
    nchunk = rows // _CH
    mesh = plsc.VectorSubcoreMesh(core_axis_name="c", subcore_axis_name="s")

    @functools.partial(
        pl.kernel,
        out_type=jax.ShapeDtypeStruct((B, S, D), table.dtype),
        mesh=mesh,
        scratch_types=(
            [pltpu.VMEM((_CH, D), table.dtype) for _ in range(_NBUF)]
            + [pltpu.SemaphoreType.DMA, pltpu.SemaphoreType.DMA]
        ),
    )
    def body(table_hbm, out_hbm, *rest):
        bufs, (in_sem, out_sem) = list(rest[:_NBUF]), rest[_NBUF:]
        wid = lax.axis_index("s") * _NC + lax.axis_index("c")
        base = wid * rows

        def start_in(i):
            return pltpu.async_copy(
                table_hbm.at[pl.ds(base + i * _CH, _CH)],
                bufs[i % _NBUF], in_sem)

        in_copies = [None] * nchunk
        out_copies = [None] * nchunk
        drained = [False] * nchunk
        in_copies[0] = start_in(0)
        for i in range(nchunk):
            in_copies[i].wait()
            out_copies[i] = [
                pltpu.async_copy(
                    bufs[i % _NBUF],
                    out_hbm.at[b].at[pl.ds(base + i * _CH, _CH)],
                    out_sem)
                for b in range(B)
            ]
            if i + 1 < nchunk:
                prev_user = i + 1 - _NBUF
                if prev_user >= 0:
                    for c in out_copies[prev_user]:
                        c.wait()
                    drained[prev_user] = True
                in_copies[i + 1] = start_in(i + 1)
        for i in range(nchunk):
            if not drained[i]:
                for c in out_copies[i]:
                    c.wait()

    return body(table)


def kernel(x, table):
    B, S, D = x.shape
    return _sc_copy(table, B, S, D)


# speedup vs baseline: 3.2254x; 1.4597x over previous
import functools

import jax
import jax.numpy as jnp
from jax import lax
from jax.experimental import pallas as pl
from jax.experimental.pallas import tpu as pltpu
from jax.experimental.pallas import tpu_sc as plsc

_NC = 2
_NS = 16
_CH = 32
_NBUF = 3
_TC_CHUNK = 1024


def _tc_body(t_ref, o_ref):
    o_ref[...] = jnp.broadcast_to(t_ref[...][None], o_ref.shape)


def _tc_copy(table, B, S, D):
    return pl.pallas_call(
        _tc_body,
        grid=(S // _TC_CHUNK,),
        in_specs=[pl.BlockSpec((_TC_CHUNK, D), lambda i: (i, 0))],
        out_specs=pl.BlockSpec((B, _TC_CHUNK, D), lambda i: (0, i, 0)),
        out_shape=jax.ShapeDtypeStruct((B, S, D), table.dtype),
    )(table)


def _sc_copy(table, B, S, D):
    NW = _NC * _NS
    rows = S // NW
    nchunk = rows // _CH
    mesh = plsc.VectorSubcoreMesh(core_axis_name="c", subcore_axis_name="s")

    @functools.partial(
        pl.kernel,
        out_type=jax.ShapeDtypeStruct((B, S, D), table.dtype),
        mesh=mesh,
        scratch_types=(
            [pltpu.VMEM((_CH, D), table.dtype) for _ in range(_NBUF)]
            + [pltpu.SemaphoreType.DMA, pltpu.SemaphoreType.DMA]
        ),
    )
    def body(table_hbm, out_hbm, *rest):
        bufs, (in_sem, out_sem) = list(rest[:_NBUF]), rest[_NBUF:]
        wid = lax.axis_index("s") * _NC + lax.axis_index("c")
        base = wid * rows

        def start_in(i):
            return pltpu.async_copy(
                table_hbm.at[pl.ds(base + i * _CH, _CH)],
                bufs[i % _NBUF], in_sem)

        in_copies = [None] * nchunk
        out_copies = [None] * nchunk
        drained = [False] * nchunk
        in_copies[0] = start_in(0)
        for i in range(nchunk):
            in_copies[i].wait()
            out_copies[i] = [
                pltpu.async_copy(
                    bufs[i % _NBUF],
                    out_hbm.at[b].at[pl.ds(base + i * _CH, _CH)],
                    out_sem)
                for b in range(B)
            ]
            if i + 1 < nchunk:
                prev_user = i + 1 - _NBUF
                if prev_user >= 0:
                    for c in out_copies[prev_user]:
                        c.wait()
                    drained[prev_user] = True
                in_copies[i + 1] = start_in(i + 1)
        for i in range(nchunk):
            if not drained[i]:
                for c in out_copies[i]:
                    c.wait()

    return body(table)


def kernel(x, table):
    B, S, D = x.shape
    return _tc_copy(table, B, S, D)
